# baseline (speedup 1.0000x reference)
import functools

import jax
import jax.numpy as jnp
from jax.experimental import pallas as pl
from jax.experimental.pallas import tpu as pltpu

HIDDEN = 2048
NUM_EXPERTS = 16
TOKENS = 16384
BLOCK = 2048


def _router_kernel(x_ref, wt_ref, w_out_ref, i_out_ref):
    logits = jnp.dot(x_ref[...], wt_ref[...],
                     preferred_element_type=jnp.float32)
    lane = jax.lax.broadcasted_iota(jnp.int32, logits.shape, 1)
    m1 = jnp.max(logits, axis=1, keepdims=True)
    i1 = jnp.min(jnp.where(logits == m1, lane, NUM_EXPERTS), axis=1,
                 keepdims=True)
    masked = jnp.where(lane == i1, -jnp.inf, logits)
    m2 = jnp.max(masked, axis=1, keepdims=True)
    i2 = jnp.min(jnp.where(masked == m2, lane, NUM_EXPERTS), axis=1,
                 keepdims=True)
    r = jnp.exp(m2 - m1)
    w1 = 1.0 / (1.0 + r)
    w_out_ref[...] = jnp.concatenate([w1, 1.0 - w1], axis=1)
    i_out_ref[...] = jnp.concatenate([i1, i2], axis=1)


@jax.jit
def kernel(x, W):
    grid = (TOKENS // BLOCK,)
    w_out, i_out = pl.pallas_call(
        _router_kernel,
        grid=grid,
        in_specs=[
            pl.BlockSpec((BLOCK, HIDDEN), lambda i: (i, 0)),
            pl.BlockSpec((HIDDEN, NUM_EXPERTS), lambda i: (0, 0)),
        ],
        out_specs=[
            pl.BlockSpec((BLOCK, 2), lambda i: (i, 0)),
            pl.BlockSpec((BLOCK, 2), lambda i: (i, 0)),
        ],
        out_shape=[
            jax.ShapeDtypeStruct((TOKENS, 2), jnp.float32),
            jax.ShapeDtypeStruct((TOKENS, 2), jnp.int32),
        ],
        compiler_params=pltpu.CompilerParams(
            dimension_semantics=("arbitrary",),
        ),
    )(x, W.T)
    return (w_out, i_out)
